# baseline (speedup 1.0000x reference)
import jax
import jax.numpy as jnp
from jax import lax
from jax.experimental import pallas as pl
from jax.experimental.pallas import tpu as pltpu
from jax.experimental.pallas import tpu_sc as plsc

_T = 8192
_H = 256
_S = 8
_D1 = 64
_BT = 4096
_NW = 32
_TPW = _T // _NW
_HALO = 8
_LANES = 16


def _tc_stage(h_ref, w1_ref, tw2_ref, ws_ref, b1c_ref, w2_ref,
              cu_ref, b2_ref, bs_ref, gps_ref):
    i = pl.program_id(0)
    h = h_ref[...]
    wc = w1_ref[...] * tw2_ref[...]
    y = lax.dot_general(wc, h, (((0,), (1,)), ((), ())),
                        preferred_element_type=jnp.float32)
    r = jnp.maximum(y + b1c_ref[...], 0.0)
    g = lax.dot_general(w2_ref[...], r, (((0,), (0,)), ((), ())),
                        preferred_element_type=jnp.float32) + b2_ref[0]
    p = lax.dot_general(ws_ref[...], h, (((0,), (1,)), ((), ())),
                        preferred_element_type=jnp.float32) + bs_ref[0]
    pos = i * _BT + lax.broadcasted_iota(jnp.int32, (1, _BT), 1)
    se = jnp.zeros((1, _BT), jnp.int32)
    for k in range(1, 9):
        se = jnp.where(pos >= cu_ref[k - 1], cu_ref[k] - 1, se)
    gps_ref[...] = jnp.concatenate([g, p, se.astype(jnp.float32)], axis=0)


def _sc_span(gps_hbm, out_hbm, g_v, p_v, se_v, out_v, sem):
    wid = lax.axis_index("s") * 2 + lax.axis_index("c")
    base = wid * _TPW
    d = jnp.where(wid == _NW - 1, 128, 0)
    cps = [
        pltpu.async_copy(gps_hbm.at[pl.ds(base - d, _TPW + 128)], g_v, sem),
        pltpu.async_copy(gps_hbm.at[pl.ds(_T + base - d, _TPW + 128)], p_v, sem),
        pltpu.async_copy(gps_hbm.at[pl.ds(2 * _T + base - d, _TPW + 128)],
                         se_v, sem),
    ]
    for cp in cps:
        cp.wait()
    lanes = lax.iota(jnp.int32, _LANES)

    @plsc.parallel_loop(0, _TPW // _LANES, step=1, unroll=4)
    def chunk(c):
        off = c * _LANES
        se = se_v[pl.ds(off + d, _LANES)]
        pos = (base + off + lanes).astype(jnp.float32)
        m = None
        ssum = None
        ws = None
        for l in range(_S):
            s_l = g_v[pl.ds(off + l + d, _LANES)]
            v_l = p_v[pl.ds(off + l + d, _LANES)]
            valid = (pos + jnp.float32(l)) <= se
            s_l = jnp.where(valid, s_l, jnp.float32(-1e30))
            v_l = jnp.where(valid, v_l, jnp.float32(0.0))
            if l == 0:
                m = s_l
                ssum = jnp.full((_LANES,), 1.0, jnp.float32)
                ws = v_l
            else:
                m2 = jnp.maximum(m, s_l)
                c1 = jnp.exp(m - m2)
                a = jnp.exp(s_l - m2)
                ssum = ssum * c1 + a
                ws = ws * c1 + a * v_l
                m = m2
            out_v[l, pl.ds(off, _LANES)] = ws / ssum

    pltpu.sync_copy(out_v, out_hbm.at[:, pl.ds(base, _TPW)])


def kernel(hidden, cu_seqlens, term_weight, W1, b1, W2, b2, Ws, bs):
    grid = _T // _BT
    gps = pl.pallas_call(
        _tc_stage,
        grid=(grid,),
        in_specs=[
            pl.BlockSpec((_BT, _H), lambda i: (i, 0)),
            pl.BlockSpec((_H, _D1), lambda i: (0, 0)),
            pl.BlockSpec((_H, 1), lambda i: (0, 0)),
            pl.BlockSpec((_H, 1), lambda i: (0, 0)),
            pl.BlockSpec((_D1, 1), lambda i: (0, 0)),
            pl.BlockSpec((_D1, 1), lambda i: (0, 0)),
            pl.BlockSpec(memory_space=pltpu.SMEM),
            pl.BlockSpec(memory_space=pltpu.SMEM),
            pl.BlockSpec(memory_space=pltpu.SMEM),
        ],
        out_specs=pl.BlockSpec((3, _BT), lambda i: (0, i)),
        out_shape=jax.ShapeDtypeStruct((3, _T), jnp.float32),
    )(hidden, W1, term_weight[:, None], Ws, b1[:, None], W2,
      cu_seqlens, b2, bs)

    sc_call = pl.kernel(
        _sc_span,
        out_type=jax.ShapeDtypeStruct((_S, _T), jnp.float32),
        mesh=plsc.VectorSubcoreMesh(core_axis_name="c", subcore_axis_name="s"),
        scratch_types=[
            pltpu.VMEM((_TPW + 128,), jnp.float32),
            pltpu.VMEM((_TPW + 128,), jnp.float32),
            pltpu.VMEM((_TPW + 128,), jnp.float32),
            pltpu.VMEM((_S, _TPW), jnp.float32),
            pltpu.SemaphoreType.DMA,
        ],
    )
    return sc_call(gps.reshape(3 * _T)).T


# speedup vs baseline: 1.1730x; 1.1730x over previous
import jax
import jax.numpy as jnp
from jax import lax
from jax.experimental import pallas as pl
from jax.experimental.pallas import tpu as pltpu
from jax.experimental.pallas import tpu_sc as plsc

_T = 8192
_H = 256
_S = 8
_D1 = 64
_BT = 4096
_NW = 32
_TPW = _T // _NW
_HALO = 8
_LANES = 16


def _tc_stage(h_ref, w1t_ref, twr_ref, wst_ref, bcol_ref, w2x_ref,
              cu_ref, b2_ref, bs_ref, gps_ref):
    i = pl.program_id(0)
    h = h_ref[...]
    wst = wst_ref[...]
    wcat = jnp.concatenate(
        [w1t_ref[...] * twr_ref[...], wst, -wst], axis=0)
    y = lax.dot_general(wcat, h, (((1,), (1,)), ((), ())),
                        preferred_element_type=jnp.float32)
    r = jnp.maximum(y + bcol_ref[...], 0.0)
    gp = jnp.dot(w2x_ref[...], r, preferred_element_type=jnp.float32)
    bias2 = jnp.concatenate(
        [jnp.full((1, 1), b2_ref[0], jnp.float32),
         jnp.full((1, 1), bs_ref[0], jnp.float32)], axis=0)
    pos = i * _BT + lax.broadcasted_iota(jnp.int32, (1, _BT), 1)
    se = jnp.zeros((1, _BT), jnp.int32)
    for k in range(1, 9):
        se = jnp.where(pos >= cu_ref[k - 1], cu_ref[k] - 1, se)
    gps_ref[...] = jnp.concatenate([gp + bias2, se.astype(jnp.float32)], axis=0)


def _sc_span(gps_hbm, out_hbm, g_v, p_v, se_v, out_v, sem):
    wid = lax.axis_index("s") * 2 + lax.axis_index("c")
    base = wid * _TPW
    d = jnp.where(wid == _NW - 1, 128, 0)
    cps = [
        pltpu.async_copy(gps_hbm.at[pl.ds(base - d, _TPW + 128)], g_v, sem),
        pltpu.async_copy(gps_hbm.at[pl.ds(_T + base - d, _TPW + 128)], p_v, sem),
        pltpu.async_copy(gps_hbm.at[pl.ds(2 * _T + base - d, _TPW + 128)],
                         se_v, sem),
    ]
    for cp in cps:
        cp.wait()
    lanes = lax.iota(jnp.int32, _LANES)

    @plsc.parallel_loop(0, _TPW // _LANES, step=1, unroll=4)
    def chunk(c):
        off = c * _LANES
        se = se_v[pl.ds(off + d, _LANES)]
        pos = (base + off + lanes).astype(jnp.float32)
        m = None
        ssum = None
        ws = None
        for l in range(_S):
            s_l = g_v[pl.ds(off + l + d, _LANES)]
            v_l = p_v[pl.ds(off + l + d, _LANES)]
            valid = (pos + jnp.float32(l)) <= se
            s_l = jnp.where(valid, s_l, jnp.float32(-1e30))
            v_l = jnp.where(valid, v_l, jnp.float32(0.0))
            if l == 0:
                m = s_l
                ssum = jnp.full((_LANES,), 1.0, jnp.float32)
                ws = v_l
            else:
                m2 = jnp.maximum(m, s_l)
                c1 = jnp.exp(m - m2)
                a = jnp.exp(s_l - m2)
                ssum = ssum * c1 + a
                ws = ws * c1 + a * v_l
                m = m2
            out_v[l, pl.ds(off, _LANES)] = ws / ssum

    pltpu.sync_copy(out_v, out_hbm.at[:, pl.ds(base, _TPW)])


def kernel(hidden, cu_seqlens, term_weight, W1, b1, W2, b2, Ws, bs):
    grid = _T // _BT
    w2x = jnp.concatenate(
        [jnp.concatenate([W2[:, 0], jnp.zeros((2,), jnp.float32)])[None, :],
         jnp.concatenate([jnp.zeros((_D1,), jnp.float32),
                          jnp.ones((1,), jnp.float32),
                          -jnp.ones((1,), jnp.float32)])[None, :]], axis=0)
    bcol = jnp.concatenate([b1, jnp.zeros((2,), jnp.float32)])[:, None]
    gps = pl.pallas_call(
        _tc_stage,
        grid=(grid,),
        in_specs=[
            pl.BlockSpec((_BT, _H), lambda i: (i, 0)),
            pl.BlockSpec((_D1, _H), lambda i: (0, 0)),
            pl.BlockSpec((1, _H), lambda i: (0, 0)),
            pl.BlockSpec((1, _H), lambda i: (0, 0)),
            pl.BlockSpec((_D1 + 2, 1), lambda i: (0, 0)),
            pl.BlockSpec((2, _D1 + 2), lambda i: (0, 0)),
            pl.BlockSpec(memory_space=pltpu.SMEM),
            pl.BlockSpec(memory_space=pltpu.SMEM),
            pl.BlockSpec(memory_space=pltpu.SMEM),
        ],
        out_specs=pl.BlockSpec((3, _BT), lambda i: (0, i)),
        out_shape=jax.ShapeDtypeStruct((3, _T), jnp.float32),
    )(hidden, W1.T, term_weight[None, :], Ws.T, bcol, w2x,
      cu_seqlens, b2, bs)

    sc_call = pl.kernel(
        _sc_span,
        out_type=jax.ShapeDtypeStruct((_S, _T), jnp.float32),
        mesh=plsc.VectorSubcoreMesh(core_axis_name="c", subcore_axis_name="s"),
        scratch_types=[
            pltpu.VMEM((_TPW + 128,), jnp.float32),
            pltpu.VMEM((_TPW + 128,), jnp.float32),
            pltpu.VMEM((_TPW + 128,), jnp.float32),
            pltpu.VMEM((_S, _TPW), jnp.float32),
            pltpu.SemaphoreType.DMA,
        ],
    )
    return sc_call(gps.reshape(3 * _T)).T
